# baseline (speedup 1.0000x reference)
import functools

import jax
import jax.numpy as jnp
from jax.experimental import pallas as pl
from jax.experimental.pallas import tpu as pltpu

T = 2048
DM = 2048
QL = 1536
H = 16
D = 128
R = 64
TOPK = 512
EPS = 1e-6
ROT = D - R
HALF = ROT // 2
FWHT_SCALE = D ** (-0.5)
WSCALE = (D ** (-0.5)) * (H ** (-0.5))
TB = 256

SENT_BASE = float(jnp.float32(-1e30))
SENT_STEP = 2.0 ** 78
SENT_THRESH = -1e29


def _bf16(x):
    return x.astype(jnp.bfloat16)


def _lane_bit(width, h):
    i = jax.lax.broadcasted_iota(jnp.int32, (1, width), 1)
    return (i & h) != 0


def _fwht(x):
    width = x.shape[-1]
    for h in (1, 2, 4, 8, 16, 32, 64):
        bit = _lane_bit(width, h)
        p = jnp.where(bit, jnp.roll(x, h, axis=1), jnp.roll(x, -h, axis=1))
        x = jnp.where(bit, p - x, x + p)
    return x


def _rope(x, cos, sin, nheads):
    width = x.shape[-1]
    ones = jnp.ones(cos.shape, jnp.float32)
    zeros = jnp.zeros(cos.shape, jnp.float32)
    c1 = jnp.concatenate([cos, cos, ones, ones], axis=1)
    c2 = jnp.concatenate([-sin, sin, zeros, zeros], axis=1)
    if nheads > 1:
        c1 = jnp.concatenate([c1] * nheads, axis=1)
        c2 = jnp.concatenate([c2] * nheads, axis=1)
    bit = _lane_bit(width, HALF)
    p = jnp.where(bit, jnp.roll(x, HALF, axis=1), jnp.roll(x, -HALF, axis=1))
    return x * c1 + p * c2


def _stage_a(k1_ref, cs_ref, kq_ref, ks_ref):
    k = k1_ref[...]
    cs = cs_ref[...]
    k = _rope(k, cs[:, :HALF], cs[:, HALF:], 1)
    k = _fwht(k) * FWHT_SCALE
    ks = jnp.maximum(jnp.max(jnp.abs(k), axis=-1, keepdims=True), 1e-4) / 448.0
    kq_ref[...] = k / ks
    ks_ref[...] = ks


def _stage_b(qq_ref, kq_ref, ks_ref, wm_ref, out_ref):
    ti = pl.program_id(0)
    qq = qq_ref[...]
    kqb = _bf16(kq_ref[...])
    ksrow = ks_ref[...]
    wm = wm_ref[...]
    parts = []
    for h in range(H):
        s = jax.lax.dot_general(_bf16(qq[:, h * D:(h + 1) * D]), kqb,
                                (((1,), (1,)), ((), ())),
                                preferred_element_type=jnp.float32)
        s = jnp.maximum(s * ksrow, 0.0)
        parts.append(_bf16(s).astype(jnp.float32)
                     * _bf16(wm[:, h][:, None]).astype(jnp.float32))
    while len(parts) > 1:
        parts = [parts[i] + parts[i + 1] for i in range(0, len(parts), 2)]
    acc = parts[0]
    t_idx = ti * TB + jax.lax.broadcasted_iota(jnp.int32, (TB, T), 0)
    s_idx = jax.lax.broadcasted_iota(jnp.int32, (TB, T), 1)
    sent = SENT_BASE - s_idx.astype(jnp.float32) * SENT_STEP
    out_ref[...] = jnp.where(s_idx <= t_idx, acc, sent)


@functools.partial(jax.jit, static_argnames=("interpret",))
def _logits(k1, q0, w0, cs, interpret=False):
    kq, ks = pl.pallas_call(
        _stage_a,
        grid=(T // TB,),
        in_specs=[
            pl.BlockSpec((TB, D), lambda i: (i, 0)),
            pl.BlockSpec((TB, ROT), lambda i: (i, 0)),
        ],
        out_specs=[
            pl.BlockSpec((TB, D), lambda i: (i, 0)),
            pl.BlockSpec((TB, 1), lambda i: (i, 0)),
        ],
        out_shape=[
            jax.ShapeDtypeStruct((T, D), jnp.float32),
            jax.ShapeDtypeStruct((T, 1), jnp.float32),
        ],
        interpret=interpret,
    )(k1, cs)

    ksrow = ks.reshape(1, T)
    logits = pl.pallas_call(
        _stage_b,
        grid=(T // TB,),
        in_specs=[
            pl.BlockSpec((TB, H * D), lambda i: (i, 0)),
            pl.BlockSpec((T, D), lambda i: (0, 0)),
            pl.BlockSpec((1, T), lambda i: (0, 0)),
            pl.BlockSpec((TB, H), lambda i: (i, 0)),
        ],
        out_specs=pl.BlockSpec((TB, T), lambda i: (i, 0)),
        out_shape=jax.ShapeDtypeStruct((T, T), jnp.float32),
        interpret=interpret,
    )(q0, kq, ksrow, w0)
    return logits


def kernel(hidden_states, q_lora, positions, wq_b_w, wk_w, k_norm_w, k_norm_b,
           weights_proj_w, cos_sin_cache):
    cs = cos_sin_cache[positions]
    cos = cs[:, :HALF]
    sin = cs[:, HALF:]
    q = (q_lora @ wq_b_w).reshape(T, H, D)
    q1 = q[..., :HALF]
    q2 = q[..., HALF:ROT]
    o1 = q1 * cos[:, None, :] - q2 * sin[:, None, :]
    o2 = q2 * cos[:, None, :] + q1 * sin[:, None, :]
    q = jnp.concatenate([o1, o2, q[..., ROT:]], axis=-1)
    orig = q.shape
    n = D
    x = q.reshape(-1, n)
    h = 1
    while h < n:
        x = x.reshape(-1, n // (2 * h), 2, h)
        a = x[:, :, 0, :]
        b = x[:, :, 1, :]
        x = jnp.concatenate([a + b, a - b], axis=-1)
        x = x.reshape(-1, n)
        h *= 2
    q = x.reshape(orig) * (D ** -0.5)
    qs = jnp.maximum(jnp.max(jnp.abs(q), axis=-1, keepdims=True), 1e-4) / 448.0
    qq = (q / qs).reshape(T, H * D)
    w0 = hidden_states @ weights_proj_w
    wm = (w0[:, :, None] * qs * ((D ** -0.5) * (H ** -0.5)))[:, :, 0]

    k0 = hidden_states @ wk_w
    mu = k0.mean(-1, keepdims=True)
    var = ((k0 - mu) ** 2).mean(-1, keepdims=True)
    k1 = (k0 - mu) / jnp.sqrt(var + EPS) * k_norm_w + k_norm_b
    logits = _logits(k1, qq, wm, cs)
    vals, idx = jax.lax.top_k(logits, TOPK)
    vals = jnp.where(vals < SENT_THRESH, SENT_BASE, vals)
    return vals, idx
